# Initial kernel scaffold; baseline (speedup 1.0000x reference)
import jax
import jax.numpy as jnp
from jax.experimental import pallas as pl


def kernel(x1, edge_index1, x2, edge_index2, W11, b11, W12, b12, W21, b21, W22, b22, W3, b3):
    raise NotImplementedError("write your pallas kernel here")


# speedup vs baseline: 4.2079x; 4.2079x over previous
import functools

import jax
import jax.numpy as jnp
from jax import lax
from jax.experimental import pallas as pl
from jax.experimental.pallas import tpu as pltpu

N = 10000
E = 160000
D = 256
H = 512
OUT = 128

EB1 = 3200
EB2 = 3200


def _leaky(t):
    return jnp.where(t >= 0, t, 0.01 * t)


def _pass1_body(dst_ref, x_ref, nodesum_ref, deg_ref):
    i = pl.program_id(0)

    @pl.when(i == 0)
    def _():
        nodesum_ref[...] = jnp.zeros_like(nodesum_ref)
        deg_ref[...] = jnp.zeros_like(deg_ref)

    dst = dst_ref[0, 0, :]
    mask = (lax.broadcasted_iota(jnp.int32, (16, EB1), 0) == dst[None, :]
            ).astype(jnp.float32)
    nodesum_ref[...] += jnp.dot(mask, x_ref[...],
                                preferred_element_type=jnp.float32)
    deg_ref[...] += jnp.sum(mask, axis=1, keepdims=True)


def _pass1(dst, x):
    nb = E // EB1
    dst3 = dst.reshape(nb, 1, EB1)
    return pl.pallas_call(
        _pass1_body,
        grid=(nb,),
        in_specs=[
            pl.BlockSpec((1, 1, EB1), lambda i: (i, 0, 0)),
            pl.BlockSpec((EB1, D), lambda i: (i, 0)),
        ],
        out_specs=[
            pl.BlockSpec((16, D), lambda i: (0, 0)),
            pl.BlockSpec((16, 1), lambda i: (0, 0)),
        ],
        out_shape=[
            jax.ShapeDtypeStruct((16, D), jnp.float32),
            jax.ShapeDtypeStruct((16, 1), jnp.float32),
        ],
    )(dst3, x)


def _z_body(nodesum_ref, deg_ref, wt_ref, z_ref):
    deg = jnp.maximum(deg_ref[...], 1.0)
    node = nodesum_ref[...] / deg
    rowmask = lax.broadcasted_iota(jnp.int32, (16, 1), 0) < 14
    node = jnp.where(rowmask, node, 0.0)
    z_ref[...] = jnp.dot(node, wt_ref[...], preferred_element_type=jnp.float32)


def _z(nodesum, deg, wt):
    din = wt.shape[0]
    return pl.pallas_call(
        _z_body,
        out_shape=jax.ShapeDtypeStruct((16, H), jnp.float32),
    )(nodesum, deg, wt)


def _p2a_body(dst_ref, c_ref, z_ref, b_ref, node2_ref):
    i = pl.program_id(0)

    @pl.when(i == 0)
    def _():
        node2_ref[...] = jnp.zeros_like(node2_ref)

    pre = jnp.dot(c_ref[...], z_ref[...],
                  preferred_element_type=jnp.float32) + b_ref[...]
    h = _leaky(pre)
    dst = dst_ref[0, 0, :]
    mask = (lax.broadcasted_iota(jnp.int32, (16, EB2), 0) == dst[None, :]
            ).astype(jnp.float32)
    node2_ref[...] += jnp.dot(mask, h, preferred_element_type=jnp.float32)


def _p2a(dst, c, z, b):
    nb = E // EB2
    dst3 = dst.reshape(nb, 1, EB2)
    return pl.pallas_call(
        _p2a_body,
        grid=(nb,),
        in_specs=[
            pl.BlockSpec((1, 1, EB2), lambda i: (i, 0, 0)),
            pl.BlockSpec((EB2, 16), lambda i: (i, 0)),
            pl.BlockSpec((16, H), lambda i: (0, 0)),
            pl.BlockSpec((1, H), lambda i: (0, 0)),
        ],
        out_specs=pl.BlockSpec((16, H), lambda i: (0, 0)),
        out_shape=jax.ShapeDtypeStruct((16, H), jnp.float32),
    )(dst3, c, z, b.reshape(1, H))


def _p2b_body(c_ref, z_ref, b_ref, colsum_ref):
    i = pl.program_id(0)

    @pl.when(i == 0)
    def _():
        colsum_ref[...] = jnp.zeros_like(colsum_ref)

    pre = jnp.dot(c_ref[...], z_ref[...],
                  preferred_element_type=jnp.float32) + b_ref[...]
    h = _leaky(pre)
    colsum_ref[...] += jnp.sum(h, axis=0, keepdims=True)


def _p2b(c, z, b):
    nb = E // EB2
    return pl.pallas_call(
        _p2b_body,
        grid=(nb,),
        in_specs=[
            pl.BlockSpec((EB2, 16), lambda i: (i, 0)),
            pl.BlockSpec((16, H), lambda i: (0, 0)),
            pl.BlockSpec((1, H), lambda i: (0, 0)),
        ],
        out_specs=pl.BlockSpec((1, H), lambda i: (0, 0)),
        out_shape=jax.ShapeDtypeStruct((1, H), jnp.float32),
    )(c, z, b.reshape(1, H))


def _final_body(s1_ref, s2_ref, w3t_ref, b3_ref, out_ref):
    hb = jnp.concatenate([s1_ref[...], s2_ref[...]], axis=1) * (1.0 / E)
    out_ref[...] = jnp.dot(hb, w3t_ref[...],
                           preferred_element_type=jnp.float32) + b3_ref[...]


def _final(s1, s2, w3t, b3):
    return pl.pallas_call(
        _final_body,
        out_shape=jax.ShapeDtypeStruct((1, OUT), jnp.float32),
    )(s1, s2, w3t, b3.reshape(1, OUT))


def _build_c(src, dst):
    src_oh = jnp.where(
        (src < 14)[:, None],
        (src[:, None] == jnp.arange(16)[None, :]).astype(jnp.float32), 0.0)
    a = jax.ops.segment_sum(src_oh, dst, num_segments=N)
    return 0.5 * (a[src] + a[dst])


def _branch(x, src, dst, wa_t, ba, wb_t, bb):
    nodesum, deg = _pass1(dst, x)
    c = _build_c(src, dst)
    z1 = _z(nodesum, deg, wa_t)
    node2 = _p2a(dst, c, z1, ba)
    z2 = _z(node2, deg, wb_t)
    return _p2b(c, z2, bb)


def kernel(x1, edge_index1, x2, edge_index2,
           W11, b11, W12, b12, W21, b21, W22, b22, W3, b3):
    s1, d1 = edge_index1[0], edge_index1[1]
    s2, d2 = edge_index2[0], edge_index2[1]
    sum1 = _branch(x1, s1, d1, W11.T, b11, W12.T, b12)
    sum2 = _branch(x2, s2, d2, W21.T, b21, W22.T, b22)
    out = _final(sum1, sum2, W3.T, b3)
    return out.reshape(OUT)
